# Initial kernel scaffold; baseline (speedup 1.0000x reference)
import jax
import jax.numpy as jnp
from jax.experimental import pallas as pl


def kernel(z, elec_table, tables):
    raise NotImplementedError("write your pallas kernel here")


# speedup vs baseline: 10.7633x; 10.7633x over previous
import jax
import jax.numpy as jnp
from jax import lax
from jax.experimental import pallas as pl
from jax.experimental.pallas import tpu as pltpu
from jax.experimental.pallas import tpu_sc as plsc

_N_ORB = 19
_MAX_E = 15
_D = 128
_N_ELEM = 96
_N_ATOMS = 10000

_NC = 2
_NS = 16
_NW = _NC * _NS

_C = 24
_BPW = 312
_NCH = 13
_EPW = _N_ELEM // _NS
_CI_PAD = 24


def _sc_body(z_hbm, ci_hbm, tabs_hbm, out_hbm, comb_hbm,
             idx_v, rows_a, zb_v, rows_b, sem_a, sem_b):
    c = lax.axis_index("c")
    s = lax.axis_index("s")
    wid = s * _NC + c

    for j in range(_EPW):
        e = s * _EPW + j
        pltpu.sync_copy(ci_hbm.at[e], idx_v)
        pltpu.async_copy(tabs_hbm.at[idx_v], rows_a, sem_a).wait()
        pltpu.sync_copy(rows_a.at[pl.ds(0, _N_ORB)], comb_hbm.at[e])
    plsc.subcore_barrier()

    base = wid * _BPW
    nch = _NCH + jnp.where(wid == _NW - 1, 1, 0)

    def chunk(k, carry):
        b = jnp.minimum(base + k * _C, _N_ATOMS - _C)
        pltpu.sync_copy(z_hbm.at[pl.ds(b, _C)], zb_v)
        pltpu.async_copy(comb_hbm.at[zb_v], rows_b, sem_b).wait()
        pltpu.sync_copy(rows_b, out_hbm.at[pl.ds(b, _C)])
        return carry

    lax.fori_loop(0, nch, chunk, None)


def kernel(z, elec_table, tables):
    z = z.astype(jnp.int32)
    tabs = tables.at[:, 0, :].set(0.0).reshape(_N_ORB * _MAX_E, _D)
    ci = elec_table.astype(jnp.int32) + (jnp.arange(_N_ORB, dtype=jnp.int32) * _MAX_E)[None, :]
    ci = jnp.pad(ci, ((0, 0), (0, _CI_PAD - _N_ORB)))

    mesh = plsc.VectorSubcoreMesh(core_axis_name="c", subcore_axis_name="s")
    out, _ = pl.kernel(
        _sc_body,
        out_type=[
            jax.ShapeDtypeStruct((_N_ATOMS, _N_ORB, _D), jnp.float32),
            jax.ShapeDtypeStruct((_N_ELEM, _N_ORB, _D), jnp.float32),
        ],
        mesh=mesh,
        scratch_types=[
            pltpu.VMEM((_CI_PAD,), jnp.int32),
            pltpu.VMEM((_CI_PAD, _D), jnp.float32),
            pltpu.VMEM((_C,), jnp.int32),
            pltpu.VMEM((_C, _N_ORB, _D), jnp.float32),
            pltpu.SemaphoreType.DMA,
            pltpu.SemaphoreType.DMA,
        ],
    )(z, ci, tabs)
    return out
